# Initial kernel scaffold; baseline (speedup 1.0000x reference)
import jax
import jax.numpy as jnp
from jax.experimental import pallas as pl


def kernel(query_length, key_length, relative_attention_bias):
    raise NotImplementedError("write your pallas kernel here")


# speedup vs baseline: 22.4250x; 22.4250x over previous
import math
import functools

import jax
import jax.numpy as jnp
import numpy as np
from jax.experimental import pallas as pl
from jax.experimental.pallas import tpu as pltpu

_NUM_BUCKETS = 32
_MAX_DISTANCE = 128
_NUM_HEADS = 16
_Q = 2048
_K = 2048
_QB = 256
_KB = 256


def _bucket_thresholds():
    d = np.arange(0, 4096)
    rp = d.astype(np.float32)
    tmp = np.log(rp / np.float32(16.0) + np.float32(1e-10))
    tmp = tmp / np.float32(math.log(_MAX_DISTANCE / 16))
    tmp = tmp * np.float32(16.0)
    large = np.minimum(16 + tmp.astype(np.int32), _NUM_BUCKETS - 1)
    b = np.where(d < 16, d, large)
    return [int(np.argmax(b >= k)) for k in range(1, _NUM_BUCKETS)]


_THRESHOLDS = _bucket_thresholds()


def _bias_kernel(doff_ref, tbl_ref, out_ref):
    i = pl.program_id(1)
    j = pl.program_id(2)
    diag = (i - j) * _QB + doff_ref[0]
    dmax = diag + (_QB - 1)
    dmin = diag - (_KB - 1)
    t1 = _THRESHOLDS[0]
    t31 = _THRESHOLDS[-1]

    v0 = tbl_ref[0, 0, 0]
    v31 = tbl_ref[0, 0, _NUM_BUCKETS - 1]

    @pl.when(dmax < t1)
    def _const_upper():
        out_ref[...] = jnp.full((1, _QB, _KB), v0, jnp.float32)

    @pl.when(dmin >= t31)
    def _const_lower():
        out_ref[...] = jnp.full((1, _QB, _KB), v31, jnp.float32)

    @pl.when((dmax >= t1) & (dmin < t31))
    def _band():
        rq = jax.lax.broadcasted_iota(jnp.int32, (1, _QB, _KB), 1)
        rk = jax.lax.broadcasted_iota(jnp.int32, (1, _QB, _KB), 2)
        d = rq - rk + diag
        acc = jnp.full((1, _QB, _KB), v0, jnp.float32)
        for b in range(1, _NUM_BUCKETS):
            delta = tbl_ref[0, 0, b] - tbl_ref[0, 0, b - 1]
            acc = acc + jnp.where(d >= _THRESHOLDS[b - 1], delta, 0.0)
        out_ref[...] = acc


def kernel(query_length, key_length, relative_attention_bias):
    q_offset = jnp.asarray(query_length, jnp.int32) - _Q
    k_offset = jnp.asarray(key_length, jnp.int32) - _K
    doff = (q_offset - k_offset).reshape(1)

    tbl = jnp.transpose(relative_attention_bias, (1, 0))[:, None, :]

    grid = (_NUM_HEADS, _Q // _QB, _K // _KB)
    out = pl.pallas_call(
        _bias_kernel,
        grid=grid,
        in_specs=[
            pl.BlockSpec(memory_space=pltpu.SMEM),
            pl.BlockSpec((1, 1, _NUM_BUCKETS), lambda h, i, j: (h, 0, 0)),
        ],
        out_specs=pl.BlockSpec((1, _QB, _KB), lambda h, i, j: (h, i, j)),
        out_shape=jax.ShapeDtypeStruct((_NUM_HEADS, _Q, _K), jnp.float32),
        compiler_params=pltpu.CompilerParams(
            dimension_semantics=("parallel", "parallel", "parallel"),
        ),
    )(doff, tbl)
    return out[None]


# speedup vs baseline: 70.4014x; 3.1394x over previous
import math

import jax
import jax.numpy as jnp
import numpy as np
from jax.experimental import pallas as pl
from jax.experimental.pallas import tpu as pltpu

_NUM_BUCKETS = 32
_MAX_DISTANCE = 128
_NUM_HEADS = 16
_Q = 2048
_K = 2048
_QB = 128
_KB = 128


def _bucket_thresholds():
    d = np.arange(0, 4096)
    rp = d.astype(np.float32)
    tmp = np.log(rp / np.float32(16.0) + np.float32(1e-10))
    tmp = tmp / np.float32(math.log(_MAX_DISTANCE / 16))
    tmp = tmp * np.float32(16.0)
    large = np.minimum(16 + tmp.astype(np.int32), _NUM_BUCKETS - 1)
    b = np.where(d < 16, d, large)
    return [int(np.argmax(b >= k)) for k in range(1, _NUM_BUCKETS)]


_THRESHOLDS = _bucket_thresholds()


def _bias_kernel(doff_ref, tbl_ref, out_ref):
    i = pl.program_id(0)
    j = pl.program_id(1)
    diag = (i - j) * _QB + doff_ref[0]
    dmax = diag + (_QB - 1)
    dmin = diag - (_KB - 1)
    t1 = _THRESHOLDS[0]
    t31 = _THRESHOLDS[-1]

    def _col(b):
        return tbl_ref[:, :, b][:, :, None]

    @pl.when(dmax < t1)
    def _const_upper():
        out_ref[...] = jnp.broadcast_to(_col(0), (_NUM_HEADS, _QB, _KB))

    @pl.when(dmin >= t31)
    def _const_lower():
        out_ref[...] = jnp.broadcast_to(
            _col(_NUM_BUCKETS - 1), (_NUM_HEADS, _QB, _KB)
        )

    @pl.when((dmax >= t1) & (dmin < t31))
    def _band():
        rq = jax.lax.broadcasted_iota(jnp.int32, (1, _QB, _KB), 1)
        rk = jax.lax.broadcasted_iota(jnp.int32, (1, _QB, _KB), 2)
        d = rq - rk + diag
        acc = jnp.broadcast_to(_col(0), (_NUM_HEADS, _QB, _KB))
        for b in range(1, _NUM_BUCKETS):
            delta = _col(b) - _col(b - 1)
            maskf = (d >= _THRESHOLDS[b - 1]).astype(jnp.float32)
            acc = acc + maskf * delta
        out_ref[...] = acc


def kernel(query_length, key_length, relative_attention_bias):
    q_offset = jnp.asarray(query_length, jnp.int32) - _Q
    k_offset = jnp.asarray(key_length, jnp.int32) - _K
    doff = (q_offset - k_offset).reshape(1)

    tbl = jnp.transpose(relative_attention_bias, (1, 0))[:, None, :]

    grid = (_Q // _QB, _K // _KB)
    out = pl.pallas_call(
        _bias_kernel,
        grid=grid,
        in_specs=[
            pl.BlockSpec(memory_space=pltpu.SMEM),
            pl.BlockSpec((_NUM_HEADS, 1, _NUM_BUCKETS), lambda i, j: (0, 0, 0)),
        ],
        out_specs=pl.BlockSpec(
            (_NUM_HEADS, _QB, _KB), lambda i, j: (0, i, j)
        ),
        out_shape=jax.ShapeDtypeStruct((_NUM_HEADS, _Q, _K), jnp.float32),
        compiler_params=pltpu.CompilerParams(
            dimension_semantics=("parallel", "parallel"),
        ),
    )(doff, tbl)
    return out[None]
